# baseline (speedup 1.0000x reference)
import functools

import jax
import jax.numpy as jnp
from jax import lax
from jax.experimental import pallas as pl
from jax.experimental.pallas import tpu as pltpu
from jax.experimental.pallas import tpu_sc as plsc

VOCAB = 100000
D = 2048
R = 64
L = 8
T = 8192

NC = 2
NS = 16
NW = NC * NS
TPW = T // NW

_SC_MESH = plsc.VectorSubcoreMesh(core_axis_name="c", subcore_axis_name="s")


@functools.partial(
    pl.kernel,
    out_type=[jax.ShapeDtypeStruct((T,), jnp.int32)],
    mesh=_SC_MESH,
    scratch_types=[
        pltpu.VMEM((TPW,), jnp.int32),
    ],
)
def _sc_floor(vids_hbm, aw_hbm, out_hbm, vids_v):
    wid = lax.axis_index("s") * NC + lax.axis_index("c")
    tbase = wid * TPW
    pltpu.sync_copy(vids_hbm.at[pl.ds(tbase, TPW)], vids_v)
    pltpu.sync_copy(vids_v, out_hbm.at[pl.ds(tbase, TPW)])


def kernel(input_, token_weight_indices, weight, embedding_A_buffer, embedding_B_buffer):
    vids = input_.astype(jnp.int32)
    aw = lax.bitcast_convert_type(
        embedding_A_buffer.astype(jnp.bfloat16).reshape(L * R * VOCAB // 2, 2),
        jnp.int32)
    (o,) = _sc_floor(vids, aw)
    return o.astype(jnp.float32)[:, None] * jnp.zeros((1, D), jnp.float32)


# speedup vs baseline: 41.0648x; 41.0648x over previous
import functools

import jax
import jax.numpy as jnp
from jax import lax
from jax.experimental import pallas as pl
from jax.experimental.pallas import tpu as pltpu
from jax.experimental.pallas import tpu_sc as plsc

VOCAB = 100000
D = 2048
R = 64
L = 8
T = 8192

NC = 2
NS = 16
NW = NC * NS
TPW = T // NW
ROW_CHUNK = 8
NBUF = 4
N_ROW_CHUNKS = TPW // ROW_CHUNK
LA_CHUNK = 128
N_LA_CHUNKS = (R * TPW) // LA_CHUNK

_SC_MESH = plsc.VectorSubcoreMesh(core_axis_name="c", subcore_axis_name="s")


@functools.partial(
    pl.kernel,
    out_type=[jax.ShapeDtypeStruct((T, D), jnp.float32)],
    mesh=_SC_MESH,
    scratch_types=[
        pltpu.VMEM((TPW,), jnp.int32),
        pltpu.VMEM((NBUF, ROW_CHUNK, D), jnp.float32),
        pltpu.SemaphoreType.DMA,
        pltpu.SemaphoreType.DMA,
        pltpu.SemaphoreType.DMA,
        pltpu.SemaphoreType.DMA,
        pltpu.SemaphoreType.DMA,
        pltpu.SemaphoreType.DMA,
        pltpu.SemaphoreType.DMA,
        pltpu.SemaphoreType.DMA,
    ],
)
def _sc_rows(weight_hbm, vids_hbm, base_hbm, vids_v, rowbuf,
             sg0, sg1, sg2, sg3, so0, so1, so2, so3):
    semg = (sg0, sg1, sg2, sg3)
    semo = (so0, so1, so2, so3)
    wid = lax.axis_index("s") * NC + lax.axis_index("c")
    tbase = wid * TPW

    pltpu.sync_copy(vids_hbm.at[pl.ds(tbase, TPW)], vids_v)

    for b in range(NBUF):
        pltpu.async_copy(
            weight_hbm.at[vids_v.at[pl.ds(b * ROW_CHUNK, ROW_CHUNK)]],
            rowbuf.at[b], semg[b])

    def _group(g, _):
        for b in range(NBUF):
            c = g * NBUF + b
            pltpu.make_async_copy(
                weight_hbm.at[pl.ds(0, ROW_CHUNK)], rowbuf.at[b], semg[b]).wait()
            pltpu.async_copy(
                rowbuf.at[b],
                base_hbm.at[pl.ds(tbase + c * ROW_CHUNK, ROW_CHUNK)],
                semo[b])
            pltpu.make_async_copy(
                rowbuf.at[b],
                base_hbm.at[pl.ds(tbase, ROW_CHUNK)],
                semo[b]).wait()

            @pl.when(c + NBUF < N_ROW_CHUNKS)
            def _():
                pltpu.async_copy(
                    weight_hbm.at[vids_v.at[pl.ds((c + NBUF) * ROW_CHUNK, ROW_CHUNK)]],
                    rowbuf.at[b], semg[b])
        return 0

    lax.fori_loop(0, N_ROW_CHUNKS // NBUF, _group, 0)


@functools.partial(
    pl.kernel,
    out_type=[jax.ShapeDtypeStruct((T * R,), jnp.float32)],
    mesh=_SC_MESH,
    scratch_types=[
        pltpu.VMEM((TPW,), jnp.int32),
        pltpu.VMEM((TPW,), jnp.int32),
        pltpu.VMEM((R * TPW,), jnp.int32),
        pltpu.VMEM((R * TPW,), jnp.float32),
        pltpu.SemaphoreType.DMA,
    ],
)
def _sc_la(a_flat_hbm, vids_hbm, tw_hbm, la_hbm,
           vids_v, tw_v, la_idx_v, la_out_v, sem_la):
    wid = lax.axis_index("s") * NC + lax.axis_index("c")
    tbase = wid * TPW

    pltpu.sync_copy(vids_hbm.at[pl.ds(tbase, TPW)], vids_v)
    pltpu.sync_copy(tw_hbm.at[pl.ds(tbase, TPW)], tw_v)

    def _tok_chunk(tc, _):
        v16 = vids_v[pl.ds(tc * 16, 16)]
        l16 = tw_v[pl.ds(tc * 16, 16)]
        base16 = l16 * (R * VOCAB) + v16

        def _row(r, _):
            la_idx_v[pl.ds(r * TPW + tc * 16, 16)] = base16 + r * VOCAB
            return 0

        lax.fori_loop(0, R, _row, 0)
        return 0

    lax.fori_loop(0, TPW // 16, _tok_chunk, 0)

    def _fire_la(c, _):
        pltpu.async_copy(
            a_flat_hbm.at[la_idx_v.at[pl.ds(c * LA_CHUNK, LA_CHUNK)]],
            la_out_v.at[pl.ds(c * LA_CHUNK, LA_CHUNK)],
            sem_la,
        )
        return 0

    lax.fori_loop(0, N_LA_CHUNKS, _fire_la, 0)

    pltpu.make_async_copy(
        a_flat_hbm.at[pl.ds(0, R * TPW)], la_out_v, sem_la).wait()
    pltpu.sync_copy(la_out_v, la_hbm.at[pl.ds(wid * (R * TPW), R * TPW)])


def _tc_body(tw_ref, base_ref, la_ref, b_ref, out_ref):
    tw = tw_ref[0]
    a_t = la_ref[0]
    lmin = jnp.min(tw)
    lmax = jnp.max(tw)
    out_ref[...] = base_ref[...]
    for l in range(L):
        @pl.when(jnp.logical_and(lmin <= l, l <= lmax))
        def _():
            m = (tw == l).astype(jnp.float32)
            am = a_t * m
            contrib = lax.dot_general(
                am, b_ref[l],
                dimension_numbers=(((0,), (1,)), ((), ())),
                preferred_element_type=jnp.float32,
            )
            out_ref[...] += contrib


def _tc_combine(tw3, base, la, b):
    return pl.pallas_call(
        _tc_body,
        grid=(NW,),
        in_specs=[
            pl.BlockSpec((1, 1, TPW), lambda i: (i, 0, 0)),
            pl.BlockSpec((TPW, D), lambda i: (i, 0)),
            pl.BlockSpec((1, R, TPW), lambda i: (i, 0, 0)),
            pl.BlockSpec((L, D, R), lambda i: (0, 0, 0)),
        ],
        out_specs=pl.BlockSpec((TPW, D), lambda i: (i, 0)),
        out_shape=jax.ShapeDtypeStruct((T, D), jnp.float32),
        compiler_params=pltpu.CompilerParams(
            dimension_semantics=("arbitrary",),
        ),
    )(tw3, base, la, b)


def kernel(input_, token_weight_indices, weight, embedding_A_buffer, embedding_B_buffer):
    vids = input_.astype(jnp.int32)
    tw = token_weight_indices.astype(jnp.int32)
    a_flat = embedding_A_buffer.reshape(-1)
    (base,) = _sc_rows(weight, vids)
    (la_flat,) = _sc_la(a_flat, vids, tw)
    la = la_flat.reshape(NW, R, TPW)
    tw3 = tw.reshape(NW, 1, TPW)
    return _tc_combine(tw3, base, la, embedding_B_buffer)
